# baseline (speedup 1.0000x reference)
import jax
import jax.numpy as jnp
from jax.experimental import pallas as pl
from jax.experimental.pallas import tpu as pltpu

_C = 2000
_NBUF = 6


def _mlp_pipe(u_hbm, w1, b1, w2, b2, o_hbm, u_buf, o_buf, in_sem, out_sem):
    n = u_hbm.shape[0]
    nchunks = n // _C

    def in_copy(i, slot):
        return pltpu.make_async_copy(
            u_hbm.at[pl.ds(i * _C, _C), :], u_buf.at[slot], in_sem.at[slot])

    def out_copy(i, slot):
        return pltpu.make_async_copy(
            o_buf.at[slot], o_hbm.at[pl.ds(i * _C, _C), :], out_sem.at[slot])

    for s in range(_NBUF):
        in_copy(s, s).start()

    def body(i, carry):
        slot = jax.lax.rem(i, _NBUF)
        in_copy(i, slot).wait()
        h = jnp.dot(u_buf[slot], w1[:], preferred_element_type=jnp.float32)
        h = h + b1[:]
        h = jnp.where(h >= 0, h, 0.2 * h)
        o = jnp.dot(h, w2[:], preferred_element_type=jnp.float32)
        o = o + b2[:]

        @pl.when(i >= _NBUF)
        def _():
            out_copy(i - _NBUF, slot).wait()

        o_buf[slot] = o
        out_copy(i, slot).start()

        @pl.when(i + _NBUF < nchunks)
        def _():
            in_copy(i + _NBUF, slot).start()

        return carry

    jax.lax.fori_loop(0, nchunks, body, 0, unroll=False)

    for s in range(_NBUF):
        i = nchunks - _NBUF + s
        out_copy(i, i % _NBUF).wait()


def kernel(u_st, W1, b1, W2, b2):
    n, d = u_st.shape
    hdim = W1.shape[0]
    return pl.pallas_call(
        _mlp_pipe,
        in_specs=[
            pl.BlockSpec(memory_space=pl.ANY),
            pl.BlockSpec(memory_space=pltpu.VMEM),
            pl.BlockSpec(memory_space=pltpu.VMEM),
            pl.BlockSpec(memory_space=pltpu.VMEM),
            pl.BlockSpec(memory_space=pltpu.VMEM),
        ],
        out_specs=pl.BlockSpec(memory_space=pl.ANY),
        out_shape=jax.ShapeDtypeStruct((n, d), jnp.float32),
        scratch_shapes=[
            pltpu.VMEM((_NBUF, _C, d), jnp.float32),
            pltpu.VMEM((_NBUF, _C, d), jnp.float32),
            pltpu.SemaphoreType.DMA((_NBUF,)),
            pltpu.SemaphoreType.DMA((_NBUF,)),
        ],
    )(u_st, W1.T, b1.reshape(1, hdim), W2.T, b2.reshape(1, d))
